# baseline (speedup 1.0000x reference)
import functools

import jax
import jax.numpy as jnp
from jax import lax
from jax.experimental import pallas as pl
from jax.experimental.pallas import tpu as pltpu
from jax.experimental.pallas import tpu_sc as plsc

K = 8
B, S, C = 4, 8192, 768
L = 16
NC, NS = 2, 16
NW = NC * NS
SGC = 96
NSG = C // SGC
NCS = SGC // L
CHUNK = 512
NCHUNK = S // CHUNK
G = 16
NGRP = CHUNK // G


def _insert(state, v):
    out = []
    for s in state:
        hi = jnp.maximum(s, v)
        v = jnp.minimum(s, v)
        out.append(hi)
    return tuple(out)


def _merge(sa, sb):
    for r in sb:
        sa = _insert(sa, r)
    return sa


def _tree_max(rows):
    m = list(rows)
    while len(m) > 1:
        m = [jnp.maximum(m[2 * i], m[2 * i + 1]) for i in range(len(m) // 2)]
    return m[0]


@functools.partial(
    pl.kernel,
    mesh=plsc.VectorSubcoreMesh(core_axis_name="c", subcore_axis_name="s"),
    out_type=jax.ShapeDtypeStruct((B, C * K), jnp.float32),
    scratch_types=[
        pltpu.VMEM((CHUNK, NCS, L), jnp.float32),
        pltpu.VMEM((CHUNK, NCS, L), jnp.float32),
        pltpu.VMEM((NGRP, L), jnp.float32),
        pltpu.VMEM((K, L), jnp.int32),
        pltpu.VMEM((NCS, 2 * K, L), jnp.float32),
        pltpu.VMEM((SGC * K,), jnp.float32),
        pltpu.SemaphoreType.DMA,
        pltpu.SemaphoreType.DMA,
    ],
    compiler_params=pltpu.CompilerParams(
        use_tc_tiling_on_sc=False, needs_layout_passes=False),
)
def _kmax_kernel(x_hbm, out_hbm, buf0, buf1, gbuf, idxbuf, stbuf, obuf,
                 sem0, sem1):
    wid = lax.axis_index("s") * NC + lax.axis_index("c")
    b = wid // NSG
    sg = wid % NSG
    lanes = lax.broadcasted_iota(jnp.int32, (L,), 0)
    neg = jnp.full((L,), -jnp.inf, dtype=jnp.float32)

    def src(ci):
        return x_hbm.at[b, pl.ds(ci * CHUNK, CHUNK), sg, :, :]

    def init_body(cs, _):
        for i in range(2 * K):
            stbuf[cs, i] = neg
        return _

    lax.fori_loop(0, NCS, init_body, 0)

    def process_slice(buf, cs):
        sa = tuple(stbuf[cs, i] for i in range(K))
        sb = tuple(stbuf[cs, K + i] for i in range(K))
        tv = jnp.maximum(sa[K - 1], sb[K - 1])
        csf = jnp.full((L,), cs, dtype=jnp.int32)

        def p1(g, cm):
            base = g * (2 * G)
            g0 = _tree_max([buf[base + r, cs] for r in range(G)])
            g1 = _tree_max([buf[base + G + r, cs] for r in range(G)])
            gbuf[2 * g] = g0
            gbuf[2 * g + 1] = g1
            return jnp.maximum(cm, jnp.maximum(g0, g1))

        cm = lax.fori_loop(0, NGRP // 2, p1, neg)
        pred = jnp.any(cm > tv)

        def do(st):
            sa, sb = st[:K], st[K:]

            def tins(g, thr):
                ta, tb = thr[:K], thr[K:]
                return (_insert(ta, gbuf[2 * g])
                        + _insert(tb, gbuf[2 * g + 1]))

            thr = lax.fori_loop(0, NGRP // 2, tins, ((neg,) * K) * 2)
            t8 = _merge(thr[:K], thr[K:])[K - 1]

            def p2_strict(g, cnt):
                gm = gbuf[g]
                m = (gm > t8) & (gm > tv) & (cnt < K)
                plsc.store_scatter(idxbuf, [cnt, lanes],
                                   jnp.full((L,), g, dtype=jnp.int32),
                                   mask=m)
                return cnt + m.astype(jnp.int32)

            cnt = lax.fori_loop(0, NGRP, p2_strict,
                                jnp.zeros((L,), jnp.int32))

            def p2_ties(g, cnt):
                gm = gbuf[g]
                m = (gm == t8) & (gm > tv) & (cnt < K)
                plsc.store_scatter(idxbuf, [cnt, lanes],
                                   jnp.full((L,), g, dtype=jnp.int32),
                                   mask=m)
                return cnt + m.astype(jnp.int32)

            cnt = lax.fori_loop(0, NGRP, p2_ties, cnt)

            def p3(j, st2):
                a, b2 = st2[:K], st2[K:]
                live = j < cnt
                rb = idxbuf[j] * G
                for r in range(K):
                    v = plsc.load_gather(buf, [rb + r, csf, lanes])
                    a = _insert(a, jnp.where(live, v, neg))
                for r in range(K, G):
                    v = plsc.load_gather(buf, [rb + r, csf, lanes])
                    b2 = _insert(b2, jnp.where(live, v, neg))
                return a + b2

            return lax.fori_loop(0, K, p3, sa + sb)

        st = lax.cond(pred, do, lambda st: st, sa + sb)
        for i in range(K):
            stbuf[cs, i] = st[i]
            stbuf[cs, K + i] = st[K + i]

    bufs = ((buf0, sem0), (buf1, sem1))
    pltpu.async_copy(src(0), buf0, sem0)

    def pair_body(p, _carry):
        for par in range(2):
            ci = p * 2 + par
            buf, sem = bufs[par]
            nbuf, nsem = bufs[1 - par]
            pltpu.make_async_copy(src(ci), buf, sem).wait()

            @pl.when(ci < NCHUNK - 1)
            def _():
                pltpu.async_copy(src(ci + 1), nbuf, nsem)

            lax.fori_loop(0, NCS, lambda cs, _c, _b=buf:
                          (process_slice(_b, cs), _c)[1], 0)
        return _carry

    lax.fori_loop(0, NCHUNK // 2, pair_body, 0)

    def emit(cs, _):
        sa = tuple(stbuf[cs, i] for i in range(K))
        sb = tuple(stbuf[cs, K + i] for i in range(K))
        state = _merge(sa, sb)
        for j in range(K):
            plsc.store_scatter(obuf, [cs * (L * K) + lanes * K + j],
                               state[j])
        return _

    lax.fori_loop(0, NCS, emit, 0)
    pltpu.sync_copy(obuf, out_hbm.at[b, pl.ds(sg * (SGC * K), SGC * K)])


def kernel(inputs):
    x = inputs.reshape(B, S, NSG, NCS, L)
    return _kmax_kernel(x)


# speedup vs baseline: 4.1896x; 4.1896x over previous
import functools

import jax
import jax.numpy as jnp
from jax import lax
from jax.experimental import pallas as pl
from jax.experimental.pallas import tpu as pltpu
from jax.experimental.pallas import tpu_sc as plsc

K = 8
B, S, C = 4, 8192, 768
L = 16
NC, NS = 2, 16
NW = NC * NS
CG = C // L
TASKS = B * CG
TPW = TASKS // NW
CHUNK = 2048
NCHUNK = S // CHUNK
G = 16
NGROUP = CHUNK // G


def _insert(state, v):
    out = []
    for s in state:
        hi = jnp.maximum(s, v)
        v = jnp.minimum(s, v)
        out.append(hi)
    return tuple(out)


def _merge(sa, sb):
    for r in sb:
        sa = _insert(sa, r)
    return sa


def _tree_max(rows):
    m = list(rows)
    while len(m) > 1:
        m = [jnp.maximum(m[2 * i], m[2 * i + 1]) for i in range(len(m) // 2)]
    return m[0]


@functools.partial(
    pl.kernel,
    mesh=plsc.VectorSubcoreMesh(core_axis_name="c", subcore_axis_name="s"),
    out_type=jax.ShapeDtypeStruct((B, C * K), jnp.float32),
    scratch_types=[
        pltpu.VMEM((CHUNK, L), jnp.float32),
        pltpu.VMEM((CHUNK, L), jnp.float32),
        pltpu.VMEM((NGROUP, L), jnp.float32),
        pltpu.VMEM((K, L), jnp.int32),
        pltpu.VMEM((L * K,), jnp.float32),
        pltpu.SemaphoreType.DMA,
        pltpu.SemaphoreType.DMA,
    ],
    compiler_params=pltpu.CompilerParams(
        use_tc_tiling_on_sc=False, needs_layout_passes=False),
)
def _kmax_kernel(x_hbm, out_hbm, buf0, buf1, gbuf, idxbuf, obuf, sem0, sem1):
    wid = lax.axis_index("s") * NC + lax.axis_index("c")
    lanes = lax.broadcasted_iota(jnp.int32, (L,), 0)
    neg = jnp.full((L,), -jnp.inf, dtype=jnp.float32)

    def src(t, ci):
        b = t // CG
        cg = t % CG
        return x_hbm.at[b, pl.ds(ci * CHUNK, CHUNK), pl.ds(cg * L, L)]

    def process_chunk(buf, sa, sb):
        tv = jnp.maximum(sa[K - 1], sb[K - 1])

        def p1(g, cm):
            base = g * (2 * G)
            g0 = _tree_max([buf[base + r] for r in range(G)])
            g1 = _tree_max([buf[base + G + r] for r in range(G)])
            gbuf[2 * g] = g0
            gbuf[2 * g + 1] = g1
            return jnp.maximum(cm, jnp.maximum(g0, g1))

        cm = lax.fori_loop(0, NGROUP // 2, p1, neg)
        pred = jnp.any(cm > tv)

        def do(st):
            sa, sb = st[:K], st[K:]

            def tins(g, thr):
                ta, tb = thr[:K], thr[K:]
                return (_insert(ta, gbuf[2 * g])
                        + _insert(tb, gbuf[2 * g + 1]))

            thr = lax.fori_loop(0, NGROUP // 2, tins, ((neg,) * K) * 2)
            t8 = _merge(thr[:K], thr[K:])[K - 1]

            def p2_strict(g, cnt):
                gm = gbuf[g]
                m = (gm > t8) & (gm > tv) & (cnt < K)
                plsc.store_scatter(idxbuf, [cnt, lanes],
                                   jnp.full((L,), g, dtype=jnp.int32),
                                   mask=m)
                return cnt + m.astype(jnp.int32)

            cnt = lax.fori_loop(0, NGROUP, p2_strict,
                                jnp.zeros((L,), jnp.int32))

            def p2_ties(g, cnt):
                gm = gbuf[g]
                m = (gm == t8) & (gm > tv) & (cnt < K)
                plsc.store_scatter(idxbuf, [cnt, lanes],
                                   jnp.full((L,), g, dtype=jnp.int32),
                                   mask=m)
                return cnt + m.astype(jnp.int32)

            cnt = lax.fori_loop(0, NGROUP, p2_ties, cnt)

            def p3(j, st2):
                a, b2 = st2[:K], st2[K:]
                live = j < cnt
                rb = idxbuf[j] * G
                for r in range(K):
                    v = plsc.load_gather(buf, [rb + r, lanes])
                    a = _insert(a, jnp.where(live, v, neg))
                for r in range(K, G):
                    v = plsc.load_gather(buf, [rb + r, lanes])
                    b2 = _insert(b2, jnp.where(live, v, neg))
                return a + b2

            return lax.fori_loop(0, K, p3, sa + sb)

        st = lax.cond(pred, do, lambda st: st, sa + sb)
        return st[:K], st[K:]

    bufs = ((buf0, sem0), (buf1, sem1))
    pltpu.async_copy(src(wid * TPW, 0), buf0, sem0)

    def task_body(it, _carry):
        t = wid * TPW + it
        b = t // CG
        cg = t % CG
        sa = (neg,) * K
        sb = (neg,) * K

        for ci in range(NCHUNK):
            buf, sem = bufs[ci % 2]
            nbuf, nsem = bufs[(ci + 1) % 2]
            pltpu.make_async_copy(src(t, ci), buf, sem).wait()
            if ci < NCHUNK - 1:
                pltpu.async_copy(src(t, ci + 1), nbuf, nsem)
            else:
                @pl.when(it < TPW - 1)
                def _():
                    pltpu.async_copy(src(t + 1, 0), nbuf, nsem)
            sa, sb = process_chunk(buf, sa, sb)

        state = _merge(sa, sb)
        for j in range(K):
            plsc.store_scatter(obuf, [lanes * K + j], state[j])
        pltpu.sync_copy(obuf, out_hbm.at[b, pl.ds(cg * (L * K), L * K)])
        return _carry

    lax.fori_loop(0, TPW, task_body, 0)


def kernel(inputs):
    return _kmax_kernel(inputs)


# speedup vs baseline: 4.4565x; 1.0637x over previous
import functools

import jax
import jax.numpy as jnp
from jax import lax
from jax.experimental import pallas as pl
from jax.experimental.pallas import tpu as pltpu
from jax.experimental.pallas import tpu_sc as plsc

K = 8
B, S, C = 4, 8192, 768
L = 16
NC, NS = 2, 16
NW = NC * NS
CG = C // L
TASKS = B * CG
TPW = TASKS // NW
CHUNK = 2048
NCHUNK = S // CHUNK
G = 16
NGROUP = CHUNK // G


def _insert(state, v):
    out = []
    for s in state:
        hi = jnp.maximum(s, v)
        v = jnp.minimum(s, v)
        out.append(hi)
    return tuple(out)


def _merge(sa, sb):
    for r in sb:
        sa = _insert(sa, r)
    return sa


def _tree_max(rows):
    m = list(rows)
    while len(m) > 1:
        m = [jnp.maximum(m[2 * i], m[2 * i + 1]) for i in range(len(m) // 2)]
    return m[0]


@functools.partial(
    pl.kernel,
    mesh=plsc.VectorSubcoreMesh(core_axis_name="c", subcore_axis_name="s"),
    out_type=jax.ShapeDtypeStruct((B, C * K), jnp.float32),
    scratch_types=[
        pltpu.VMEM((CHUNK, L), jnp.float32),
        pltpu.VMEM((CHUNK, L), jnp.float32),
        pltpu.VMEM((NGROUP, L), jnp.float32),
        pltpu.VMEM((K, L), jnp.int32),
        pltpu.VMEM((L * K,), jnp.float32),
        pltpu.SemaphoreType.DMA,
        pltpu.SemaphoreType.DMA,
    ],
    compiler_params=pltpu.CompilerParams(
        use_tc_tiling_on_sc=False, needs_layout_passes=False),
)
def _kmax_kernel(x_hbm, out_hbm, buf0, buf1, gbuf, idxbuf, obuf, sem0, sem1):
    wid = lax.axis_index("s") * NC + lax.axis_index("c")
    lanes = lax.broadcasted_iota(jnp.int32, (L,), 0)
    neg = jnp.full((L,), -jnp.inf, dtype=jnp.float32)

    def src(t, ci):
        b = t // CG
        cg = t % CG
        return x_hbm.at[b, pl.ds(ci * CHUNK, CHUNK), pl.ds(cg * L, L)]

    def process_chunk(buf, sa, sb):
        def p1(g, thr):
            ta, tb = thr[:K], thr[K:]
            base = g * (2 * G)
            g0 = _tree_max([buf[base + r] for r in range(G)])
            g1 = _tree_max([buf[base + G + r] for r in range(G)])
            gbuf[2 * g] = g0
            gbuf[2 * g + 1] = g1
            return _insert(ta, g0) + _insert(tb, g1)

        thr = lax.fori_loop(0, NGROUP // 2, p1, ((neg,) * K) * 2)
        t8 = _merge(thr[:K], thr[K:])[K - 1]

        def p2_strict(g, cnt):
            m = (gbuf[g] > t8) & (cnt < K)
            plsc.store_scatter(idxbuf, [cnt, lanes],
                               jnp.full((L,), g, dtype=jnp.int32), mask=m)
            return cnt + m.astype(jnp.int32)

        cnt = lax.fori_loop(0, NGROUP, p2_strict, jnp.zeros((L,), jnp.int32))

        def p2_ties(g, cnt):
            m = (gbuf[g] == t8) & (cnt < K)
            plsc.store_scatter(idxbuf, [cnt, lanes],
                               jnp.full((L,), g, dtype=jnp.int32), mask=m)
            return cnt + m.astype(jnp.int32)

        lax.fori_loop(0, NGROUP, p2_ties, cnt)

        def p3(j, st):
            a, b2 = st[:K], st[K:]
            rb = idxbuf[j] * G
            for r in range(G // 2):
                a = _insert(a, plsc.load_gather(buf, [rb + r, lanes]))
            for r in range(G // 2, G):
                b2 = _insert(b2, plsc.load_gather(buf, [rb + r, lanes]))
            return a + b2

        st = lax.fori_loop(0, K, p3, tuple(sa) + tuple(sb))
        return st[:K], st[K:]

    bufs = ((buf0, sem0), (buf1, sem1))
    pltpu.async_copy(src(wid * TPW, 0), buf0, sem0)

    def task_body(it, _carry):
        t = wid * TPW + it
        b = t // CG
        cg = t % CG
        sa = (neg,) * K
        sb = (neg,) * K

        for ci in range(NCHUNK):
            buf, sem = bufs[ci % 2]
            nbuf, nsem = bufs[(ci + 1) % 2]
            pltpu.make_async_copy(src(t, ci), buf, sem).wait()
            if ci < NCHUNK - 1:
                pltpu.async_copy(src(t, ci + 1), nbuf, nsem)
            else:
                @pl.when(it < TPW - 1)
                def _():
                    pltpu.async_copy(src(t + 1, 0), nbuf, nsem)
            sa, sb = process_chunk(buf, sa, sb)

        state = _merge(sa, sb)
        for j in range(K):
            plsc.store_scatter(obuf, [lanes * K + j], state[j])
        pltpu.sync_copy(obuf, out_hbm.at[b, pl.ds(cg * (L * K), L * K)])
        return _carry

    lax.fori_loop(0, TPW, task_body, 0)


def kernel(inputs):
    return _kmax_kernel(inputs)
